# speedup vs baseline: 4.5950x; 4.5950x over previous
import jax
import jax.numpy as jnp
from jax.experimental import pallas as pl
from jax.experimental.pallas import tpu as pltpu

N_NODES = 100000
D_FEAT = 128
N_GROUPS = 8
EMB = 16
BLOCK = 2000


def _body(x_ref, gt_ref, ht_ref, o_ref):
    i = pl.program_id(0)
    base = i * BLOCK
    o_ref[:, :D_FEAT] = x_ref[...]
    rid8 = jax.lax.broadcasted_iota(jnp.int32, (BLOCK, N_GROUPS), 0) + base
    gid = jnp.where((rid8 % 100 == 0) & (rid8 < 8000), rid8 // 1000, 0)
    col = jax.lax.broadcasted_iota(jnp.int32, (BLOCK, N_GROUPS), 1)
    onehot = (gid == col).astype(jnp.float32)
    o_ref[:, D_FEAT:D_FEAT + EMB] = jnp.dot(
        onehot, gt_ref[...], preferred_element_type=jnp.float32)
    rid16 = jax.lax.broadcasted_iota(jnp.int32, (BLOCK, EMB), 0) + base
    o_ref[:, D_FEAT + EMB:] = jnp.where(
        (rid16 & 1) == 0, ht_ref[0:1, :], ht_ref[1:2, :])


def kernel(x, group_table, hemi_table):
    n = x.shape[0]
    grid = n // BLOCK
    return pl.pallas_call(
        _body,
        grid=(grid,),
        in_specs=[
            pl.BlockSpec((BLOCK, D_FEAT), lambda i: (i, 0)),
            pl.BlockSpec((N_GROUPS, EMB), lambda i: (0, 0)),
            pl.BlockSpec((2, EMB), lambda i: (0, 0)),
        ],
        out_specs=pl.BlockSpec((BLOCK, D_FEAT + 2 * EMB), lambda i: (i, 0)),
        out_shape=jax.ShapeDtypeStruct((n, D_FEAT + 2 * EMB), jnp.float32),
        compiler_params=pltpu.CompilerParams(
            dimension_semantics=("arbitrary",),
        ),
    )(x, group_table, hemi_table)


# speedup vs baseline: 5.7580x; 1.2531x over previous
import jax
import jax.numpy as jnp
from jax.experimental import pallas as pl
from jax.experimental.pallas import tpu as pltpu

N_NODES = 100000
D_FEAT = 128
N_GROUPS = 8
EMB = 16
BLOCK = 4000
SPECIAL_BLOCKS = -(-8000 // BLOCK)


def _body(x_ref, gt_ref, ht_ref, o_ref):
    i = pl.program_id(0)
    o_ref[:, :D_FEAT] = x_ref[...]
    par = jax.lax.broadcasted_iota(jnp.int32, (BLOCK, EMB), 0) & 1
    o_ref[:, D_FEAT + EMB:] = jnp.where(par == 0, ht_ref[0:1, :], ht_ref[1:2, :])

    @pl.when(i >= SPECIAL_BLOCKS)
    def _steady():
        o_ref[:, D_FEAT:D_FEAT + EMB] = jnp.broadcast_to(
            gt_ref[0:1, :], (BLOCK, EMB))

    @pl.when(i < SPECIAL_BLOCKS)
    def _special():
        base = i * BLOCK
        rid8 = jax.lax.broadcasted_iota(jnp.int32, (BLOCK, N_GROUPS), 0) + base
        gid = jnp.where((rid8 % 100 == 0) & (rid8 < 8000), rid8 // 1000, 0)
        col = jax.lax.broadcasted_iota(jnp.int32, (BLOCK, N_GROUPS), 1)
        onehot = (gid == col).astype(jnp.float32)
        o_ref[:, D_FEAT:D_FEAT + EMB] = jnp.dot(
            onehot, gt_ref[...], preferred_element_type=jnp.float32)


def kernel(x, group_table, hemi_table):
    n = x.shape[0]
    grid = n // BLOCK
    return pl.pallas_call(
        _body,
        grid=(grid,),
        in_specs=[
            pl.BlockSpec((BLOCK, D_FEAT), lambda i: (i, 0)),
            pl.BlockSpec((N_GROUPS, EMB), lambda i: (0, 0)),
            pl.BlockSpec((2, EMB), lambda i: (0, 0)),
        ],
        out_specs=pl.BlockSpec((BLOCK, D_FEAT + 2 * EMB), lambda i: (i, 0)),
        out_shape=jax.ShapeDtypeStruct((n, D_FEAT + 2 * EMB), jnp.float32),
        compiler_params=pltpu.CompilerParams(
            dimension_semantics=("arbitrary",),
        ),
    )(x, group_table, hemi_table)


# speedup vs baseline: 5.9929x; 1.0408x over previous
import jax
import jax.numpy as jnp
from jax.experimental import pallas as pl
from jax.experimental.pallas import tpu as pltpu

N_NODES = 100000
D_FEAT = 128
N_GROUPS = 8
EMB = 16
ENC = 2 * EMB

SPECIAL_ROWS = 8000
CHUNK = 2000
NCHUNK = N_NODES // CHUNK
SPECIAL_CHUNKS = SPECIAL_ROWS // CHUNK
NBUF = 8
OUTLAG = 4


def _hemi_block(ht_ref):
    par = jax.lax.broadcasted_iota(jnp.int32, (CHUNK, EMB), 0) & 1
    return jnp.where(par == 0, ht_ref[0:1, :], ht_ref[1:2, :])


def _group_block(base, gt_ref):
    rid = jax.lax.broadcasted_iota(jnp.int32, (CHUNK, N_GROUPS), 0) + base
    gid = jnp.where((rid % 100 == 0) & (rid < SPECIAL_ROWS), rid // 1000, 0)
    col = jax.lax.broadcasted_iota(jnp.int32, (CHUNK, N_GROUPS), 1)
    onehot = (gid == col).astype(jnp.float32)
    return jnp.dot(onehot, gt_ref[...], preferred_element_type=jnp.float32)


def _body(x_hbm, gt_ref, ht_ref, o_hbm, obuf, insem, outsem):
    def start_in(j):
        s = j % NBUF
        pltpu.make_async_copy(
            x_hbm.at[pl.ds(j * CHUNK, CHUNK), :],
            obuf.at[s].at[:, pl.ds(0, D_FEAT)],
            insem.at[s]).start()

    def wait_in(j):
        s = j % NBUF
        pltpu.make_async_copy(
            x_hbm.at[pl.ds(j * CHUNK, CHUNK), :],
            obuf.at[s].at[:, pl.ds(0, D_FEAT)],
            insem.at[s]).wait()

    def start_out(j):
        s = j % NBUF
        pltpu.make_async_copy(
            obuf.at[s], o_hbm.at[pl.ds(j * CHUNK, CHUNK), :],
            outsem.at[s]).start()

    def wait_out(j):
        s = j % NBUF
        pltpu.make_async_copy(
            obuf.at[s], o_hbm.at[pl.ds(j * CHUNK, CHUNK), :],
            outsem.at[s]).wait()

    hemi = _hemi_block(ht_ref)
    steady = jnp.broadcast_to(gt_ref[0:1, :], (CHUNK, EMB))
    for s in range(NBUF):
        obuf[s, :, D_FEAT:D_FEAT + EMB] = steady
        obuf[s, :, D_FEAT + EMB:] = hemi

    for j in range(NBUF):
        start_in(j)

    for k in range(NCHUNK):
        wait_in(k)
        if k < SPECIAL_CHUNKS:
            obuf[k % NBUF, :, D_FEAT:D_FEAT + EMB] = _group_block(
                k * CHUNK, gt_ref)
        elif NBUF <= k < NBUF + SPECIAL_CHUNKS:
            obuf[k % NBUF, :, D_FEAT:D_FEAT + EMB] = steady
        start_out(k)
        r = k - OUTLAG
        if r >= 0:
            wait_out(r)
            if r + NBUF < NCHUNK:
                start_in(r + NBUF)

    for r in range(max(0, NCHUNK - OUTLAG), NCHUNK):
        wait_out(r)


def kernel(x, group_table, hemi_table):
    n = x.shape[0]
    return pl.pallas_call(
        _body,
        in_specs=[
            pl.BlockSpec(memory_space=pl.ANY),
            pl.BlockSpec(memory_space=pltpu.VMEM),
            pl.BlockSpec(memory_space=pltpu.VMEM),
        ],
        out_specs=pl.BlockSpec(memory_space=pl.ANY),
        out_shape=jax.ShapeDtypeStruct((n, D_FEAT + ENC), jnp.float32),
        scratch_shapes=[
            pltpu.VMEM((NBUF, CHUNK, D_FEAT + ENC), jnp.float32),
            pltpu.SemaphoreType.DMA((NBUF,)),
            pltpu.SemaphoreType.DMA((NBUF,)),
        ],
    )(x, group_table, hemi_table)
